# baseline (speedup 1.0000x reference)
import functools

import jax
import jax.numpy as jnp
from jax import lax
from jax.experimental import pallas as pl
from jax.experimental.pallas import tpu as pltpu
from jax.experimental.pallas import tpu_sc as plsc

_GRID = 4
_WIN = 3
_NC = 2
_NS = 16
_NW = _NC * _NS
_NBUF = 4
_SPLIT = 2
_ROWS_PER_ITER = 2


def _fuser_body(units_per_worker, gh, gw, n_row_tiles, n_col_tiles, K,
                smp_hbm, ref_hbm, sr_hbm, out_hbm, srv, *rest):
    ibufs = rest[0:_NBUF]
    obufs = rest[_NBUF:2 * _NBUF]
    isems = rest[2 * _NBUF:3 * _NBUF]
    osems = rest[3 * _NBUF:4 * _NBUF]
    uh = gh // _SPLIT

    cid = lax.axis_index("c")
    sid = lax.axis_index("s")
    wid = sid * _NC + cid
    base = wid * units_per_worker
    units_per_b = n_row_tiles * n_col_tiles * _SPLIT
    b = base // units_per_b
    rem0 = base % units_per_b

    pltpu.sync_copy(sr_hbm.at[pl.ds(b * 2 * K, 16)], srv)
    sv = srv[...]
    max_r = n_row_tiles - _WIN
    max_c = n_col_tiles - _WIN
    rks = [jnp.clip(sv[2 * k], 0, max_r) for k in range(K)]
    cks = [jnp.clip(sv[2 * k + 1], 0, max_c) for k in range(K)]

    def unit_state(u):
        rem = rem0 + u
        tile = rem // _SPLIT
        half = rem % _SPLIT
        ti = tile // n_col_tiles
        tj = tile % n_col_tiles
        kk = jnp.int32(-1)
        for k in range(K):
            cov = ((ti >= rks[k]) & (ti < rks[k] + _WIN)
                   & (tj >= cks[k]) & (tj < cks[k] + _WIN))
            kk = jnp.where(cov, jnp.int32(k), kk)
        return kk, ti * gh + half * uh, tj * gw

    def start_in(u):
        kk, y, x = unit_state(u)
        buf = ibufs[u % _NBUF]
        sem = isems[u % _NBUF]

        @pl.when(kk >= 0)
        def _():
            pltpu.async_copy(
                ref_hbm.at[b, kk, pl.ds(y, uh), pl.ds(x, gw)], buf, sem)

        @pl.when(kk < 0)
        def _():
            pltpu.async_copy(
                smp_hbm.at[b, pl.ds(y, uh), pl.ds(x, gw)], buf, sem)

    def wait_in(u):
        pltpu.make_async_copy(
            smp_hbm.at[0, pl.ds(0, uh), pl.ds(0, gw)],
            ibufs[u % _NBUF], isems[u % _NBUF]).wait()

    def start_out(u):
        _, y, x = unit_state(u)
        pltpu.async_copy(
            obufs[u % _NBUF],
            out_hbm.at[b, pl.ds(y, uh), pl.ds(x, gw)], osems[u % _NBUF])

    def wait_out(u):
        _, y, x = unit_state(u)
        pltpu.make_async_copy(
            obufs[u % _NBUF],
            out_hbm.at[b, pl.ds(y, uh), pl.ds(x, gw)], osems[u % _NBUF]).wait()

    def sigmoid_unit(src, dst):
        def rows(r2, c2):
            r0 = r2 * _ROWS_PER_ITER
            for dr in range(_ROWS_PER_ITER):
                for c in range(gw // 16):
                    v = src[r0 + dr, pl.ds(c * 16, 16)]
                    dst[r0 + dr, pl.ds(c * 16, 16)] = \
                        1.0 / (1.0 + jnp.exp(-v))
            return c2

        lax.fori_loop(0, uh // _ROWS_PER_ITER, rows, 0)

    for u in range(min(_NBUF, units_per_worker)):
        start_in(u)
    for u in range(units_per_worker):
        wait_in(u)
        if u >= _NBUF:
            wait_out(u - _NBUF)
        sigmoid_unit(ibufs[u % _NBUF], obufs[u % _NBUF])
        start_out(u)
        if u + _NBUF < units_per_worker:
            start_in(u + _NBUF)
    for u in range(max(0, units_per_worker - _NBUF), units_per_worker):
        wait_out(u)


def kernel(sampling_map, refined_response_maps, selected_regions):
    B, C, H, W = sampling_map.shape
    K = refined_response_maps.shape[1]
    gh = H // _GRID
    gw = W // _GRID

    sr = jnp.pad(selected_regions.reshape(B * K * 2), (0, 16))

    smp = sampling_map.reshape(B, H, W)
    ref = refined_response_maps.reshape(B, K, H, W)

    n_units = B * _GRID * _GRID * _SPLIT
    units_per_worker = n_units // _NW

    mesh = plsc.VectorSubcoreMesh(core_axis_name="c", subcore_axis_name="s")
    body = functools.partial(_fuser_body, units_per_worker, gh, gw,
                             _GRID, _GRID, K)
    fn = pl.kernel(
        body,
        out_type=jax.ShapeDtypeStruct((B, H, W), jnp.float32),
        mesh=mesh,
        scratch_types=(
            [pltpu.VMEM((16,), jnp.int32)]
            + [pltpu.VMEM((gh // _SPLIT, gw), jnp.float32)] * (2 * _NBUF)
            + [pltpu.SemaphoreType.DMA] * (2 * _NBUF)
        ),
    )
    out = fn(smp, ref, sr)
    return out.reshape(B, C, H, W)


# speedup vs baseline: 1.0645x; 1.0645x over previous
import functools

import jax
import jax.numpy as jnp
from jax import lax
from jax.experimental import pallas as pl
from jax.experimental.pallas import tpu as pltpu
from jax.experimental.pallas import tpu_sc as plsc

_GRID = 4
_WIN = 3
_NC = 2
_NS = 16
_NW = _NC * _NS
_NBUF = 3
_ROWS_PER_ITER = 2


def _fuser_body(tiles_per_worker, gh, gw, n_row_tiles, n_col_tiles, K,
                smp_hbm, ref_hbm, sr_hbm, out_hbm, srv, *rest):
    ibufs = rest[0:_NBUF]
    obufs = rest[_NBUF:2 * _NBUF]
    isems = rest[2 * _NBUF:3 * _NBUF]
    osems = rest[3 * _NBUF:4 * _NBUF]

    cid = lax.axis_index("c")
    sid = lax.axis_index("s")
    wid = sid * _NC + cid
    base = wid * tiles_per_worker
    tiles_per_b = n_row_tiles * n_col_tiles
    b = base // tiles_per_b
    rem0 = base % tiles_per_b

    pltpu.sync_copy(sr_hbm.at[pl.ds((b // 2) * (4 * K), 16)], srv)
    sv = srv[...]
    odd = (b % 2) == 1
    max_r = n_row_tiles - _WIN
    max_c = n_col_tiles - _WIN
    rks = [jnp.clip(jnp.where(odd, sv[2 * K + 2 * k], sv[2 * k]), 0, max_r)
           for k in range(K)]
    cks = [jnp.clip(jnp.where(odd, sv[2 * K + 2 * k + 1], sv[2 * k + 1]),
                    0, max_c)
           for k in range(K)]

    def tile_state(t):
        rem = rem0 + t
        ti = rem // n_col_tiles
        tj = rem % n_col_tiles
        kk = jnp.int32(-1)
        for k in range(K):
            cov = ((ti >= rks[k]) & (ti < rks[k] + _WIN)
                   & (tj >= cks[k]) & (tj < cks[k] + _WIN))
            kk = jnp.where(cov, jnp.int32(k), kk)
        return kk, ti * gh, tj * gw

    def start_in(t):
        kk, y, x = tile_state(t)
        buf = ibufs[t % _NBUF]
        sem = isems[t % _NBUF]

        @pl.when(kk >= 0)
        def _():
            pltpu.async_copy(
                ref_hbm.at[b, kk, 0, pl.ds(y, gh), pl.ds(x, gw)], buf, sem)

        @pl.when(kk < 0)
        def _():
            pltpu.async_copy(
                smp_hbm.at[b, 0, pl.ds(y, gh), pl.ds(x, gw)], buf, sem)

    def wait_in(t):
        pltpu.make_async_copy(
            smp_hbm.at[0, 0, pl.ds(0, gh), pl.ds(0, gw)],
            ibufs[t % _NBUF], isems[t % _NBUF]).wait()

    def start_out(t):
        _, y, x = tile_state(t)
        pltpu.async_copy(
            obufs[t % _NBUF],
            out_hbm.at[b, 0, pl.ds(y, gh), pl.ds(x, gw)], osems[t % _NBUF])

    def wait_out(t):
        _, y, x = tile_state(t)
        pltpu.make_async_copy(
            obufs[t % _NBUF],
            out_hbm.at[b, 0, pl.ds(y, gh), pl.ds(x, gw)],
            osems[t % _NBUF]).wait()

    def sigmoid_tile(src, dst):
        def rows(r2, c2):
            r0 = r2 * _ROWS_PER_ITER
            for dr in range(_ROWS_PER_ITER):
                for c in range(gw // 16):
                    v = src[r0 + dr, pl.ds(c * 16, 16)]
                    dst[r0 + dr, pl.ds(c * 16, 16)] = \
                        1.0 / (1.0 + jnp.exp(-v))
            return c2

        lax.fori_loop(0, gh // _ROWS_PER_ITER, rows, 0)

    for t in range(min(_NBUF, tiles_per_worker)):
        start_in(t)
    for t in range(tiles_per_worker):
        wait_in(t)
        if t >= _NBUF:
            wait_out(t - _NBUF)
        sigmoid_tile(ibufs[t % _NBUF], obufs[t % _NBUF])
        start_out(t)
        if t + _NBUF < tiles_per_worker:
            start_in(t + _NBUF)
    for t in range(max(0, tiles_per_worker - _NBUF), tiles_per_worker):
        wait_out(t)


def kernel(sampling_map, refined_response_maps, selected_regions):
    B, C, H, W = sampling_map.shape
    K = refined_response_maps.shape[1]
    gh = H // _GRID
    gw = W // _GRID

    sr = selected_regions.reshape(B * K * 2)

    n_tiles = B * _GRID * _GRID
    tiles_per_worker = n_tiles // _NW

    mesh = plsc.VectorSubcoreMesh(core_axis_name="c", subcore_axis_name="s")
    body = functools.partial(_fuser_body, tiles_per_worker, gh, gw,
                             _GRID, _GRID, K)
    fn = pl.kernel(
        body,
        out_type=jax.ShapeDtypeStruct((B, C, H, W), jnp.float32),
        mesh=mesh,
        scratch_types=(
            [pltpu.VMEM((16,), jnp.int32)]
            + [pltpu.VMEM((gh, gw), jnp.float32)] * (2 * _NBUF)
            + [pltpu.SemaphoreType.DMA] * (2 * _NBUF)
        ),
    )
    return fn(sampling_map, refined_response_maps, sr)
